# Initial kernel scaffold; baseline (speedup 1.0000x reference)
import jax
import jax.numpy as jnp
from jax.experimental import pallas as pl


def kernel(x, edge_index, W, att_src, att_dst, bias):
    raise NotImplementedError("write your pallas kernel here")


# speedup vs baseline: 54.9420x; 54.9420x over previous
import functools

import jax
import jax.numpy as jnp
from jax import lax
from jax.experimental import pallas as pl
from jax.experimental.pallas import tpu as pltpu
from jax.experimental.pallas import tpu_sc as plsc

F32 = jnp.float32
I32 = jnp.int32

NC = 2
NS = 16
NW = NC * NS
L = 16

CH = 80


def _proj_body(x_ref, w_ref, p_ref, h_ref, a_ref):
    h = jnp.dot(x_ref[...], w_ref[...], preferred_element_type=F32)
    h_ref[...] = h
    a_ref[...] = jnp.dot(h, p_ref[...], preferred_element_type=F32)


def _combine_body(num_ref, den_ref, r_ref, b_ref, o_ref):
    num = num_ref[0] + num_ref[1]
    den = den_ref[0] + den_ref[1]
    denb = jnp.dot(den, r_ref[...], preferred_element_type=F32)
    o_ref[...] = jnp.maximum(num / (denb + 1e-16) + b_ref[...], 0.0)


def _edge_body(n_nodes, e_per_w, h_hbm, a_hbm, src_hbm, dst_hbm,
               num_out, den_out, acc_num, acc_den,
               src_v, dst_v, asrc_v, adst_v, hrows_v, den_v,
               s1, s2, s3):
    cid = lax.axis_index("c")
    sid = lax.axis_index("s")
    wid = sid * NC + cid
    n_chunks = e_per_w // CH

    rpt = (n_nodes // NS) & ~7
    base_r = pl.multiple_of(sid * rpt, 8)

    def _for_my_rows(fn):
        for j in range(7):
            fn(pl.multiple_of(base_r + j * CH, 8), CH)

        @pl.when(sid < NS - 1)
        def _():
            fn(pl.multiple_of(base_r + 7 * CH, 8), rpt - 7 * CH)

        @pl.when(sid == NS - 1)
        def _():
            fn(pl.multiple_of(base_r + 7 * CH, 8), CH)

    def _zero_row(r, _):
        for j in range(8):
            hrows_v[r, pl.ds(j * L, L)] = jnp.zeros((L,), F32)
        den_v[r, :] = jnp.zeros((L,), F32)
        return 0
    lax.fori_loop(0, CH, _zero_row, 0)

    def _zero_acc(r0, nr):
        pltpu.sync_copy(hrows_v.at[pl.ds(0, nr)], acc_num.at[pl.ds(r0, nr)])
        pltpu.sync_copy(den_v.at[pl.ds(0, nr)], acc_den.at[pl.ds(r0, nr)])

    _for_my_rows(_zero_acc)
    plsc.subcore_barrier()

    lanes = lax.broadcasted_iota(I32, (L,), 0)

    def _chunk(c, _):
        base = pl.multiple_of(wid * e_per_w + c * CH, 8)
        pltpu.sync_copy(src_hbm.at[pl.ds(base, CH)], src_v)
        pltpu.sync_copy(dst_hbm.at[pl.ds(base, CH)], dst_v)
        cp1 = pltpu.async_copy(a_hbm.at[src_v], asrc_v, s1)
        cp2 = pltpu.async_copy(a_hbm.at[dst_v], adst_v, s2)
        cp3 = pltpu.async_copy(h_hbm.at[src_v], hrows_v, s3)
        cp1.wait()
        cp2.wait()
        for g in range(CH // L):
            rows16 = lanes + (g * L)
            for hd in range(4):
                s16 = plsc.load_gather(
                    asrc_v, [rows16, jnp.full((L,), hd, I32)])
                d16 = plsc.load_gather(
                    adst_v, [rows16, jnp.full((L,), 4 + hd, I32)])
                e16 = s16 + d16
                w16 = jnp.exp(jnp.maximum(e16, 0.2 * e16))
                plsc.store_scatter(
                    den_v, [rows16, jnp.full((L,), hd, I32)], w16)
        cp3.wait()

        def _scale(e, _):
            wvec = den_v[e, :]
            for hd in range(4):
                w = wvec[hd]
                for j in range(2):
                    sl = pl.ds(hd * 32 + j * L, L)
                    hrows_v[e, sl] = hrows_v[e, sl] * w
            return 0
        lax.fori_loop(0, CH, _scale, 0)

        pltpu.sync_copy(hrows_v, acc_num.at[dst_v], add=True)
        pltpu.sync_copy(den_v, acc_den.at[dst_v], add=True)
        return 0

    lax.fori_loop(0, n_chunks, _chunk, 0)
    plsc.subcore_barrier()

    def _copy_out(r0, nr):
        pltpu.sync_copy(acc_num.at[pl.ds(r0, nr)],
                        num_out.at[cid].at[pl.ds(r0, nr)])
        pltpu.sync_copy(acc_den.at[pl.ds(r0, nr)],
                        den_out.at[cid].at[pl.ds(r0, nr)])

    _for_my_rows(_copy_out)


def _run_proj(x, W, p):
    n, in_dim = x.shape
    hc = W.shape[1]
    bn = 1000
    return pl.pallas_call(
        _proj_body,
        grid=(n // bn,),
        in_specs=[
            pl.BlockSpec((bn, in_dim), lambda i: (i, 0)),
            pl.BlockSpec((in_dim, hc), lambda i: (0, 0)),
            pl.BlockSpec((in_dim, 16), lambda i: (0, 0)),
        ],
        out_specs=[
            pl.BlockSpec((bn, hc), lambda i: (i, 0)),
            pl.BlockSpec((bn, 16), lambda i: (i, 0)),
        ],
        out_shape=[
            jax.ShapeDtypeStruct((n, hc), F32),
            jax.ShapeDtypeStruct((n, 16), F32),
        ],
    )(x, W, p)


def _run_edges(h_arr, a_arr, src, dst):
    n, hc = h_arr.shape
    e_per_w = src.shape[0] // NW
    mesh = plsc.VectorSubcoreMesh(core_axis_name="c", subcore_axis_name="s")
    return pl.kernel(
        functools.partial(_edge_body, n, e_per_w),
        out_type=[
            jax.ShapeDtypeStruct((NC, n, hc), F32),
            jax.ShapeDtypeStruct((NC, n, 16), F32),
        ],
        mesh=mesh,
        compiler_params=pltpu.CompilerParams(
            use_tc_tiling_on_sc=False, needs_layout_passes=False),
        scratch_types=[
            pltpu.VMEM_SHARED((n, hc), F32),
            pltpu.VMEM_SHARED((n, 16), F32),
            pltpu.VMEM((CH,), I32),
            pltpu.VMEM((CH,), I32),
            pltpu.VMEM((CH, 16), F32),
            pltpu.VMEM((CH, 16), F32),
            pltpu.VMEM((CH, hc), F32),
            pltpu.VMEM((CH, 16), F32),
            pltpu.SemaphoreType.DMA,
            pltpu.SemaphoreType.DMA,
            pltpu.SemaphoreType.DMA,
        ],
    )(h_arr, a_arr, src, dst)


def _run_combine(num_part, den_part, r16, bias):
    _, n, hc = num_part.shape
    bn = 1000
    return pl.pallas_call(
        _combine_body,
        grid=(n // bn,),
        in_specs=[
            pl.BlockSpec((NC, bn, hc), lambda i: (0, i, 0)),
            pl.BlockSpec((NC, bn, 16), lambda i: (0, i, 0)),
            pl.BlockSpec((16, hc), lambda i: (0, 0)),
            pl.BlockSpec((1, hc), lambda i: (0, 0)),
        ],
        out_specs=pl.BlockSpec((bn, hc), lambda i: (i, 0)),
        out_shape=jax.ShapeDtypeStruct((n, hc), F32),
    )(num_part, den_part, r16, bias.reshape(1, hc))


def kernel(x, edge_index, W, att_src, att_dst, bias):
    h_heads, c_dim = att_src.shape
    hc = h_heads * c_dim

    eye = jnp.eye(h_heads, dtype=F32)
    p_src = jnp.einsum("hc,hk->hck", att_src, eye).reshape(hc, h_heads)
    p_dst = jnp.einsum("hc,hk->hck", att_dst, eye).reshape(hc, h_heads)
    p = jnp.concatenate(
        [p_src, p_dst, jnp.zeros((hc, 16 - 2 * h_heads), F32)], axis=1)

    h_arr, a_arr = _run_proj(x, W, p)
    h_arr, a_arr, src, dst = lax.optimization_barrier(
        (h_arr, a_arr, edge_index[0], edge_index[1]))
    num_part, den_part = _run_edges(h_arr, a_arr, src, dst)
    num_part, den_part = lax.optimization_barrier((num_part, den_part))

    r16 = (jnp.arange(hc)[None, :] // c_dim
           == jnp.arange(16)[:, None]).astype(F32)
    return _run_combine(num_part, den_part, r16, bias)


# speedup vs baseline: 94.0474x; 1.7118x over previous
import functools

import jax
import jax.numpy as jnp
from jax import lax
from jax.experimental import pallas as pl
from jax.experimental.pallas import tpu as pltpu
from jax.experimental.pallas import tpu_sc as plsc

F32 = jnp.float32
I32 = jnp.int32

NC = 2
NS = 16
NW = NC * NS
L = 16

CH = 80


def _proj_body(x_ref, w_ref, p_ref, h_ref, a_ref):
    h = jnp.dot(x_ref[...], w_ref[...], preferred_element_type=F32)
    h_ref[...] = h
    a_ref[...] = jnp.dot(h, p_ref[...], preferred_element_type=F32)


def _combine_body(num_ref, den_ref, r_ref, b_ref, o_ref):
    num = num_ref[0] + num_ref[1]
    den = den_ref[0] + den_ref[1]
    denb = jnp.dot(den, r_ref[...], preferred_element_type=F32)
    o_ref[...] = jnp.maximum(num / (denb + 1e-16) + b_ref[...], 0.0)


def _edge_body(n_nodes, e_per_w, h_hbm, a_hbm, src_hbm, dst_hbm,
               num_out, den_out, acc_num, acc_den,
               src_v, dst_v, dsc_v, asrc_v, adst_v, hrows_v, den_v,
               sa0, sa1, sd0, sd1, sh0, sh1, si0, si1):
    sa, sd, sh = (sa0, sa1), (sd0, sd1), (sh0, sh1)
    si = (si0, si1)
    cid = lax.axis_index("c")
    sid = lax.axis_index("s")
    wid = sid * NC + cid
    n_chunks = e_per_w // CH

    rpt = (n_nodes // NS) & ~7
    base_r = pl.multiple_of(sid * rpt, 8)

    def _for_my_rows(fn):
        for j in range(7):
            fn(pl.multiple_of(base_r + j * CH, 8), CH)

        @pl.when(sid < NS - 1)
        def _():
            fn(pl.multiple_of(base_r + 7 * CH, 8), rpt - 7 * CH)

        @pl.when(sid == NS - 1)
        def _():
            fn(pl.multiple_of(base_r + 7 * CH, 8), CH)

    def _zero_row(r, _):
        for j in range(8):
            hrows_v[0, r, pl.ds(j * L, L)] = jnp.zeros((L,), F32)
        den_v[0, r, :] = jnp.zeros((L,), F32)
        den_v[1, r, :] = jnp.zeros((L,), F32)
        return 0
    lax.fori_loop(0, CH, _zero_row, 0)

    def _zero_acc(r0, nr):
        pltpu.sync_copy(hrows_v.at[0, pl.ds(0, nr)], acc_num.at[pl.ds(r0, nr)])
        pltpu.sync_copy(den_v.at[0, pl.ds(0, nr)], acc_den.at[pl.ds(r0, nr)])

    _for_my_rows(_zero_acc)
    plsc.subcore_barrier()

    lanes = lax.broadcasted_iota(I32, (L,), 0)

    def _async_idx(c, b):
        off = pl.multiple_of(c * CH, 8)
        pltpu.async_copy(src_hbm.at[wid, pl.ds(off, CH)], src_v.at[b], si[b])
        pltpu.async_copy(dst_hbm.at[wid, pl.ds(off, CH)], dst_v.at[b], si[b])

    def _fire(c, b):
        off = pl.multiple_of(c * CH, 8)
        pltpu.make_async_copy(src_hbm.at[wid, pl.ds(off, CH)],
                              src_v.at[b], si[b]).wait()
        pltpu.make_async_copy(dst_hbm.at[wid, pl.ds(off, CH)],
                              dst_v.at[b], si[b]).wait()
        pltpu.async_copy(a_hbm.at[src_v.at[b]], asrc_v.at[b], sa[b])
        pltpu.async_copy(a_hbm.at[dst_v.at[b]], adst_v.at[b], sd[b])
        pltpu.async_copy(h_hbm.at[src_v.at[b]], hrows_v.at[b], sh[b])

    def _process(c, b, pre):
        pltpu.make_async_copy(a_hbm.at[src_v.at[b]], asrc_v.at[b],
                              sa[b]).wait()
        pltpu.make_async_copy(a_hbm.at[dst_v.at[b]], adst_v.at[b],
                              sd[b]).wait()
        for g in range(CH // L):
            rows16 = lanes + (g * L)
            for hd in range(4):
                s16 = plsc.load_gather(
                    asrc_v.at[b], [rows16, jnp.full((L,), hd, I32)])
                d16 = plsc.load_gather(
                    adst_v.at[b], [rows16, jnp.full((L,), 4 + hd, I32)])
                e16 = s16 + d16
                w16 = jnp.exp(jnp.maximum(e16, 0.2 * e16))
                plsc.store_scatter(
                    den_v.at[b], [rows16, jnp.full((L,), hd, I32)], w16)
        pltpu.make_async_copy(h_hbm.at[src_v.at[b]], hrows_v.at[b],
                              sh[b]).wait()

        for g in range(CH // L):
            dsc_v[b, pl.ds(g * L, L)] = dst_v[b, pl.ds(g * L, L)]

        @pl.when(jnp.asarray(pre, I32) < n_chunks)
        def _():
            _async_idx(pre, b)

        def _scale(e, _):
            wvec = den_v[b, e, :]
            for hd in range(4):
                w = wvec[hd]
                for j in range(2):
                    sl2 = pl.ds(hd * 32 + j * L, L)
                    hrows_v[b, e, sl2] = hrows_v[b, e, sl2] * w
            return 0
        lax.fori_loop(0, CH, _scale, 0)

        pltpu.sync_copy(hrows_v.at[b], acc_num.at[dsc_v.at[b]], add=True)
        pltpu.sync_copy(den_v.at[b], acc_den.at[dsc_v.at[b]], add=True)

    _async_idx(0, 0)
    _async_idx(1, 1)
    _fire(0, 0)

    def _pair(i, _):
        c0 = i * 2
        _fire(c0 + 1, 1)
        _process(c0, 0, c0 + 2)
        _fire(c0 + 2, 0)
        _process(c0 + 1, 1, c0 + 3)
        return 0

    lax.fori_loop(0, (n_chunks - 1) // 2, _pair, 0)
    _process(n_chunks - 1, 0, n_chunks)
    plsc.subcore_barrier()

    def _copy_out(r0, nr):
        pltpu.sync_copy(acc_num.at[pl.ds(r0, nr)],
                        num_out.at[cid].at[pl.ds(r0, nr)])
        pltpu.sync_copy(acc_den.at[pl.ds(r0, nr)],
                        den_out.at[cid].at[pl.ds(r0, nr)])

    _for_my_rows(_copy_out)


def _run_proj(x, W, p):
    n, in_dim = x.shape
    hc = W.shape[1]
    bn = 1000
    return pl.pallas_call(
        _proj_body,
        grid=(n // bn,),
        in_specs=[
            pl.BlockSpec((bn, in_dim), lambda i: (i, 0)),
            pl.BlockSpec((in_dim, hc), lambda i: (0, 0)),
            pl.BlockSpec((in_dim, 16), lambda i: (0, 0)),
        ],
        out_specs=[
            pl.BlockSpec((bn, hc), lambda i: (i, 0)),
            pl.BlockSpec((bn, 16), lambda i: (i, 0)),
        ],
        out_shape=[
            jax.ShapeDtypeStruct((n, hc), F32),
            jax.ShapeDtypeStruct((n, 16), F32),
        ],
    )(x, W, p)


def _run_edges(h_arr, a_arr, src, dst):
    n, hc = h_arr.shape
    e_per_w = src.shape[0] // NW
    n_chunks = e_per_w // CH
    mesh = plsc.VectorSubcoreMesh(core_axis_name="c", subcore_axis_name="s")
    return pl.kernel(
        functools.partial(_edge_body, n, e_per_w),
        out_type=[
            jax.ShapeDtypeStruct((NC, n, hc), F32),
            jax.ShapeDtypeStruct((NC, n, 16), F32),
        ],
        mesh=mesh,
        compiler_params=pltpu.CompilerParams(
            use_tc_tiling_on_sc=False, needs_layout_passes=False),
        scratch_types=[
            pltpu.VMEM_SHARED((n, hc), F32),
            pltpu.VMEM_SHARED((n, 16), F32),
            pltpu.VMEM((2, CH), I32),
            pltpu.VMEM((2, CH), I32),
            pltpu.VMEM((2, CH), I32),
            pltpu.VMEM((2, CH, 16), F32),
            pltpu.VMEM((2, CH, 16), F32),
            pltpu.VMEM((2, CH, hc), F32),
            pltpu.VMEM((2, CH, 16), F32),
        ] + [pltpu.SemaphoreType.DMA] * 8,
    )(h_arr, a_arr, src.reshape(NW, e_per_w), dst.reshape(NW, e_per_w))


def _run_combine(num_part, den_part, r16, bias):
    _, n, hc = num_part.shape
    bn = 1000
    return pl.pallas_call(
        _combine_body,
        grid=(n // bn,),
        in_specs=[
            pl.BlockSpec((NC, bn, hc), lambda i: (0, i, 0)),
            pl.BlockSpec((NC, bn, 16), lambda i: (0, i, 0)),
            pl.BlockSpec((16, hc), lambda i: (0, 0)),
            pl.BlockSpec((1, hc), lambda i: (0, 0)),
        ],
        out_specs=pl.BlockSpec((bn, hc), lambda i: (i, 0)),
        out_shape=jax.ShapeDtypeStruct((n, hc), F32),
    )(num_part, den_part, r16, bias.reshape(1, hc))


def kernel(x, edge_index, W, att_src, att_dst, bias):
    h_heads, c_dim = att_src.shape
    hc = h_heads * c_dim

    eye = jnp.eye(h_heads, dtype=F32)
    p_src = jnp.einsum("hc,hk->hck", att_src, eye).reshape(hc, h_heads)
    p_dst = jnp.einsum("hc,hk->hck", att_dst, eye).reshape(hc, h_heads)
    p = jnp.concatenate(
        [p_src, p_dst, jnp.zeros((hc, 16 - 2 * h_heads), F32)], axis=1)

    h_arr, a_arr = _run_proj(x, W, p)
    h_arr, a_arr, src, dst = lax.optimization_barrier(
        (h_arr, a_arr, edge_index[0], edge_index[1]))
    num_part, den_part = _run_edges(h_arr, a_arr, src, dst)
    num_part, den_part = lax.optimization_barrier((num_part, den_part))

    r16 = (jnp.arange(hc)[None, :] // c_dim
           == jnp.arange(16)[:, None]).astype(F32)
    return _run_combine(num_part, den_part, r16, bias)


# speedup vs baseline: 112.1714x; 1.1927x over previous
import functools

import jax
import jax.numpy as jnp
from jax import lax
from jax.experimental import pallas as pl
from jax.experimental.pallas import tpu as pltpu
from jax.experimental.pallas import tpu_sc as plsc

F32 = jnp.float32
I32 = jnp.int32

NC = 2
NS = 16
NW = NC * NS
L = 16

CH = 80


def _proj_body(x_ref, w_ref, p_ref, h_ref, a_ref):
    h = jnp.dot(x_ref[...], w_ref[...], preferred_element_type=F32)
    h_ref[...] = h
    a_ref[...] = jnp.dot(h, p_ref[...], preferred_element_type=F32)


def _combine_body(num_ref, den_ref, r_ref, b_ref, o_ref):
    num = num_ref[0] + num_ref[1]
    den = den_ref[0] + den_ref[1]
    denb = jnp.dot(den, r_ref[...], preferred_element_type=F32)
    o_ref[...] = jnp.maximum(num / (denb + 1e-16) + b_ref[...], 0.0)


def _edge_body(n_nodes, e_per_w, h_hbm, a_hbm, src_hbm, dst_hbm,
               num_out, den_out, acc_num, acc_den,
               src_v, dst_v, dsc_v, asrc_v, adst_v, hrows_v, den_v,
               sa0, sa1, sd0, sd1, sh0, sh1, si0, si1):
    sa, sd, sh = (sa0, sa1), (sd0, sd1), (sh0, sh1)
    si = (si0, si1)
    cid = lax.axis_index("c")
    sid = lax.axis_index("s")
    wid = sid * NC + cid
    n_chunks = e_per_w // CH

    rpt = (n_nodes // NS) & ~7
    base_r = pl.multiple_of(sid * rpt, 8)

    def _for_my_rows(fn):
        for j in range(7):
            fn(pl.multiple_of(base_r + j * CH, 8), CH)

        @pl.when(sid < NS - 1)
        def _():
            fn(pl.multiple_of(base_r + 7 * CH, 8), rpt - 7 * CH)

        @pl.when(sid == NS - 1)
        def _():
            fn(pl.multiple_of(base_r + 7 * CH, 8), CH)

    def _zero_row(r, _):
        for j in range(8):
            hrows_v[0, r, pl.ds(j * L, L)] = jnp.zeros((L,), F32)
        den_v[0, r, :] = jnp.zeros((L,), F32)
        den_v[1, r, :] = jnp.zeros((L,), F32)
        return 0
    lax.fori_loop(0, CH, _zero_row, 0)

    def _zero_acc(r0, nr):
        pltpu.sync_copy(hrows_v.at[0, pl.ds(0, nr)], acc_num.at[pl.ds(r0, nr)])
        pltpu.sync_copy(den_v.at[0, pl.ds(0, nr)], acc_den.at[pl.ds(r0, nr)])

    _for_my_rows(_zero_acc)
    plsc.subcore_barrier()

    lanes = lax.broadcasted_iota(I32, (L,), 0)

    def _async_idx(c, b):
        off = pl.multiple_of(c * CH, 8)
        pltpu.async_copy(src_hbm.at[wid, pl.ds(off, CH)], src_v.at[b], si[b])
        pltpu.async_copy(dst_hbm.at[wid, pl.ds(off, CH)], dst_v.at[b], si[b])

    def _fire(c, b):
        off = pl.multiple_of(c * CH, 8)
        pltpu.make_async_copy(src_hbm.at[wid, pl.ds(off, CH)],
                              src_v.at[b], si[b]).wait()
        pltpu.make_async_copy(dst_hbm.at[wid, pl.ds(off, CH)],
                              dst_v.at[b], si[b]).wait()
        pltpu.async_copy(a_hbm.at[src_v.at[b]], asrc_v.at[b], sa[b])
        pltpu.async_copy(a_hbm.at[dst_v.at[b]], adst_v.at[b], sd[b])
        pltpu.async_copy(h_hbm.at[src_v.at[b]], hrows_v.at[b], sh[b])

    def _process(c, b, pre):
        pltpu.make_async_copy(a_hbm.at[src_v.at[b]], asrc_v.at[b],
                              sa[b]).wait()
        pltpu.make_async_copy(a_hbm.at[dst_v.at[b]], adst_v.at[b],
                              sd[b]).wait()
        for g in range(CH // L):
            rows16 = lanes + (g * L)
            for hd in range(4):
                s16 = plsc.load_gather(
                    asrc_v.at[b], [rows16, jnp.full((L,), hd, I32)])
                d16 = plsc.load_gather(
                    adst_v.at[b], [rows16, jnp.full((L,), 4 + hd, I32)])
                e16 = s16 + d16
                w16 = jnp.exp(jnp.maximum(e16, 0.2 * e16))
                plsc.store_scatter(
                    den_v.at[b], [rows16, jnp.full((L,), hd, I32)], w16)
        pltpu.make_async_copy(h_hbm.at[src_v.at[b]], hrows_v.at[b],
                              sh[b]).wait()

        for g in range(CH // L):
            dsc_v[b, pl.ds(g * L, L)] = dst_v[b, pl.ds(g * L, L)]

        @pl.when(jnp.asarray(pre, I32) < n_chunks)
        def _():
            _async_idx(pre, b)

        @plsc.parallel_loop(0, CH, step=1, unroll=4)
        def _scale(e):
            wvec = den_v[b, e, :]
            for hd in range(4):
                w = wvec[hd]
                for j in range(2):
                    sl2 = pl.ds(hd * 32 + j * L, L)
                    hrows_v[b, e, sl2] = hrows_v[b, e, sl2] * w

        pltpu.sync_copy(hrows_v.at[b], acc_num.at[dsc_v.at[b]], add=True)
        pltpu.sync_copy(den_v.at[b], acc_den.at[dsc_v.at[b]], add=True)

    _async_idx(0, 0)
    _async_idx(1, 1)
    _fire(0, 0)

    def _pair(i, _):
        c0 = i * 2
        _fire(c0 + 1, 1)
        _process(c0, 0, c0 + 2)
        _fire(c0 + 2, 0)
        _process(c0 + 1, 1, c0 + 3)
        return 0

    lax.fori_loop(0, (n_chunks - 1) // 2, _pair, 0)
    _process(n_chunks - 1, 0, n_chunks)
    plsc.subcore_barrier()

    def _copy_out(r0, nr):
        pltpu.sync_copy(acc_num.at[pl.ds(r0, nr)],
                        num_out.at[cid].at[pl.ds(r0, nr)])
        pltpu.sync_copy(acc_den.at[pl.ds(r0, nr)],
                        den_out.at[cid].at[pl.ds(r0, nr)])

    _for_my_rows(_copy_out)


def _run_proj(x, W, p):
    n, in_dim = x.shape
    hc = W.shape[1]
    bn = 1000
    return pl.pallas_call(
        _proj_body,
        grid=(n // bn,),
        in_specs=[
            pl.BlockSpec((bn, in_dim), lambda i: (i, 0)),
            pl.BlockSpec((in_dim, hc), lambda i: (0, 0)),
            pl.BlockSpec((in_dim, 16), lambda i: (0, 0)),
        ],
        out_specs=[
            pl.BlockSpec((bn, hc), lambda i: (i, 0)),
            pl.BlockSpec((bn, 16), lambda i: (i, 0)),
        ],
        out_shape=[
            jax.ShapeDtypeStruct((n, hc), F32),
            jax.ShapeDtypeStruct((n, 16), F32),
        ],
    )(x, W, p)


def _run_edges(h_arr, a_arr, src, dst):
    n, hc = h_arr.shape
    e_per_w = src.shape[0] // NW
    n_chunks = e_per_w // CH
    mesh = plsc.VectorSubcoreMesh(core_axis_name="c", subcore_axis_name="s")
    return pl.kernel(
        functools.partial(_edge_body, n, e_per_w),
        out_type=[
            jax.ShapeDtypeStruct((NC, n, hc), F32),
            jax.ShapeDtypeStruct((NC, n, 16), F32),
        ],
        mesh=mesh,
        compiler_params=pltpu.CompilerParams(
            use_tc_tiling_on_sc=False, needs_layout_passes=False),
        scratch_types=[
            pltpu.VMEM_SHARED((n, hc), F32),
            pltpu.VMEM_SHARED((n, 16), F32),
            pltpu.VMEM((2, CH), I32),
            pltpu.VMEM((2, CH), I32),
            pltpu.VMEM((2, CH), I32),
            pltpu.VMEM((2, CH, 16), F32),
            pltpu.VMEM((2, CH, 16), F32),
            pltpu.VMEM((2, CH, hc), F32),
            pltpu.VMEM((2, CH, 16), F32),
        ] + [pltpu.SemaphoreType.DMA] * 8,
    )(h_arr, a_arr, src.reshape(NW, e_per_w), dst.reshape(NW, e_per_w))


def _run_combine(num_part, den_part, r16, bias):
    _, n, hc = num_part.shape
    bn = 1000
    return pl.pallas_call(
        _combine_body,
        grid=(n // bn,),
        in_specs=[
            pl.BlockSpec((NC, bn, hc), lambda i: (0, i, 0)),
            pl.BlockSpec((NC, bn, 16), lambda i: (0, i, 0)),
            pl.BlockSpec((16, hc), lambda i: (0, 0)),
            pl.BlockSpec((1, hc), lambda i: (0, 0)),
        ],
        out_specs=pl.BlockSpec((bn, hc), lambda i: (i, 0)),
        out_shape=jax.ShapeDtypeStruct((n, hc), F32),
    )(num_part, den_part, r16, bias.reshape(1, hc))


def kernel(x, edge_index, W, att_src, att_dst, bias):
    h_heads, c_dim = att_src.shape
    hc = h_heads * c_dim

    eye = jnp.eye(h_heads, dtype=F32)
    p_src = jnp.einsum("hc,hk->hck", att_src, eye).reshape(hc, h_heads)
    p_dst = jnp.einsum("hc,hk->hck", att_dst, eye).reshape(hc, h_heads)
    p = jnp.concatenate(
        [p_src, p_dst, jnp.zeros((hc, 16 - 2 * h_heads), F32)], axis=1)

    h_arr, a_arr = _run_proj(x, W, p)
    h_arr, a_arr, src, dst = lax.optimization_barrier(
        (h_arr, a_arr, edge_index[0], edge_index[1]))
    num_part, den_part = _run_edges(h_arr, a_arr, src, dst)
    num_part, den_part = lax.optimization_barrier((num_part, den_part))

    r16 = (jnp.arange(hc)[None, :] // c_dim
           == jnp.arange(16)[:, None]).astype(F32)
    return _run_combine(num_part, den_part, r16, bias)


# speedup vs baseline: 122.4973x; 1.0921x over previous
import functools

import jax
import jax.numpy as jnp
from jax import lax
from jax.experimental import pallas as pl
from jax.experimental.pallas import tpu as pltpu
from jax.experimental.pallas import tpu_sc as plsc

F32 = jnp.float32
I32 = jnp.int32

NC = 2
NS = 16
NW = NC * NS
L = 16

CH = 80


def _proj_body(x_ref, w_ref, p_ref, h_ref, a_ref):
    h = jnp.dot(x_ref[...], w_ref[...], preferred_element_type=F32)
    h_ref[...] = h
    a_ref[...] = jnp.dot(h, p_ref[...], preferred_element_type=F32)


def _combine_body(num_ref, den_ref, r_ref, b_ref, o_ref):
    num = num_ref[0] + num_ref[1]
    den = den_ref[0] + den_ref[1]
    denb = jnp.dot(den, r_ref[...], preferred_element_type=F32)
    o_ref[...] = jnp.maximum(num / (denb + 1e-16) + b_ref[...], 0.0)


def _edge_body(n_nodes, e_per_w, h_hbm, a_hbm, src_hbm, dst_hbm,
               num_out, den_out, acc_num, acc_den,
               src_v, dst_v, dsc_v, asrc_v, adst_v, hrows_v, hs_v, dens_v,
               sa0, sa1, sd0, sd1, sh0, sh1, si0, si1, ssn, ssd):
    sa, sd, sh = (sa0, sa1), (sd0, sd1), (sh0, sh1)
    si = (si0, si1)
    cid = lax.axis_index("c")
    sid = lax.axis_index("s")
    wid = sid * NC + cid
    n_chunks = e_per_w // CH

    rpt = (n_nodes // NS) & ~7
    base_r = pl.multiple_of(sid * rpt, 8)

    def _for_my_rows(fn):
        for j in range(7):
            fn(pl.multiple_of(base_r + j * CH, 8), CH)

        @pl.when(sid < NS - 1)
        def _():
            fn(pl.multiple_of(base_r + 7 * CH, 8), rpt - 7 * CH)

        @pl.when(sid == NS - 1)
        def _():
            fn(pl.multiple_of(base_r + 7 * CH, 8), CH)

    def _zero_row(r, _):
        for j in range(8):
            hrows_v[0, r, pl.ds(j * L, L)] = jnp.zeros((L,), F32)
        dens_v[0, r, :] = jnp.zeros((L,), F32)
        dens_v[1, r, :] = jnp.zeros((L,), F32)
        return 0
    lax.fori_loop(0, CH, _zero_row, 0)

    def _zero_acc(r0, nr):
        pltpu.sync_copy(hrows_v.at[0, pl.ds(0, nr)], acc_num.at[pl.ds(r0, nr)])
        pltpu.sync_copy(dens_v.at[0, pl.ds(0, nr)], acc_den.at[pl.ds(r0, nr)])

    _for_my_rows(_zero_acc)
    plsc.subcore_barrier()

    lanes = lax.broadcasted_iota(I32, (L,), 0)

    def _async_idx(c, b):
        off = pl.multiple_of(c * CH, 8)
        pltpu.async_copy(src_hbm.at[wid, pl.ds(off, CH)], src_v.at[b], si[b])
        pltpu.async_copy(dst_hbm.at[wid, pl.ds(off, CH)], dst_v.at[b], si[b])

    def _fire(c, b):
        off = pl.multiple_of(c * CH, 8)
        pltpu.make_async_copy(src_hbm.at[wid, pl.ds(off, CH)],
                              src_v.at[b], si[b]).wait()
        pltpu.make_async_copy(dst_hbm.at[wid, pl.ds(off, CH)],
                              dst_v.at[b], si[b]).wait()
        pltpu.async_copy(a_hbm.at[src_v.at[b]], asrc_v.at[b], sa[b])
        pltpu.async_copy(a_hbm.at[dst_v.at[b]], adst_v.at[b], sd[b])
        pltpu.async_copy(h_hbm.at[src_v.at[b]], hrows_v.at[b], sh[b])

    def _process(c, b, pre):
        pltpu.make_async_copy(a_hbm.at[src_v.at[b]], asrc_v.at[b],
                              sa[b]).wait()
        pltpu.make_async_copy(a_hbm.at[dst_v.at[b]], adst_v.at[b],
                              sd[b]).wait()

        @pl.when(jnp.asarray(c, I32) >= 2)
        def _():
            pltpu.make_async_copy(dens_v.at[b], acc_den.at[dsc_v.at[b]],
                                  ssd).wait()

        for g in range(CH // L):
            rows16 = lanes + (g * L)
            for hd in range(4):
                s16 = plsc.load_gather(
                    asrc_v.at[b], [rows16, jnp.full((L,), hd, I32)])
                d16 = plsc.load_gather(
                    adst_v.at[b], [rows16, jnp.full((L,), 4 + hd, I32)])
                e16 = s16 + d16
                w16 = jnp.exp(jnp.maximum(e16, 0.2 * e16))
                plsc.store_scatter(
                    dens_v.at[b], [rows16, jnp.full((L,), hd, I32)], w16)
        pltpu.make_async_copy(h_hbm.at[src_v.at[b]], hrows_v.at[b],
                              sh[b]).wait()

        @pl.when(jnp.asarray(c, I32) >= 1)
        def _():
            pltpu.make_async_copy(hs_v, acc_num.at[dsc_v.at[b]], ssn).wait()

        for g in range(CH // L):
            dsc_v[b, pl.ds(g * L, L)] = dst_v[b, pl.ds(g * L, L)]

        @pl.when(jnp.asarray(pre, I32) < n_chunks)
        def _():
            _async_idx(pre, b)

        @plsc.parallel_loop(0, CH, step=1, unroll=4)
        def _scale(e):
            wvec = dens_v[b, e, :]
            for hd in range(4):
                w = wvec[hd]
                for j in range(2):
                    sl2 = pl.ds(hd * 32 + j * L, L)
                    hs_v[e, sl2] = hrows_v[b, e, sl2] * w

        pltpu.async_copy(hs_v, acc_num.at[dsc_v.at[b]], ssn, add=True)
        pltpu.async_copy(dens_v.at[b], acc_den.at[dsc_v.at[b]], ssd,
                         add=True)

    _async_idx(0, 0)
    _async_idx(1, 1)
    _fire(0, 0)

    def _pair(i, _):
        c0 = i * 2
        _fire(c0 + 1, 1)
        _process(c0, 0, c0 + 2)
        _fire(c0 + 2, 0)
        _process(c0 + 1, 1, c0 + 3)
        return 0

    lax.fori_loop(0, (n_chunks - 1) // 2, _pair, 0)
    _process(n_chunks - 1, 0, n_chunks)
    pltpu.make_async_copy(hs_v, acc_num.at[dsc_v.at[0]], ssn).wait()
    pltpu.make_async_copy(dens_v.at[1], acc_den.at[dsc_v.at[1]], ssd).wait()
    pltpu.make_async_copy(dens_v.at[0], acc_den.at[dsc_v.at[0]], ssd).wait()
    plsc.subcore_barrier()

    def _copy_out(r0, nr):
        pltpu.sync_copy(acc_num.at[pl.ds(r0, nr)],
                        num_out.at[cid].at[pl.ds(r0, nr)])
        pltpu.sync_copy(acc_den.at[pl.ds(r0, nr)],
                        den_out.at[cid].at[pl.ds(r0, nr)])

    _for_my_rows(_copy_out)


def _run_proj(x, W, p):
    n, in_dim = x.shape
    hc = W.shape[1]
    bn = 1000
    return pl.pallas_call(
        _proj_body,
        grid=(n // bn,),
        in_specs=[
            pl.BlockSpec((bn, in_dim), lambda i: (i, 0)),
            pl.BlockSpec((in_dim, hc), lambda i: (0, 0)),
            pl.BlockSpec((in_dim, 16), lambda i: (0, 0)),
        ],
        out_specs=[
            pl.BlockSpec((bn, hc), lambda i: (i, 0)),
            pl.BlockSpec((bn, 16), lambda i: (i, 0)),
        ],
        out_shape=[
            jax.ShapeDtypeStruct((n, hc), F32),
            jax.ShapeDtypeStruct((n, 16), F32),
        ],
    )(x, W, p)


def _run_edges(h_arr, a_arr, src, dst):
    n, hc = h_arr.shape
    e_per_w = src.shape[0] // NW
    n_chunks = e_per_w // CH
    mesh = plsc.VectorSubcoreMesh(core_axis_name="c", subcore_axis_name="s")
    return pl.kernel(
        functools.partial(_edge_body, n, e_per_w),
        out_type=[
            jax.ShapeDtypeStruct((NC, n, hc), F32),
            jax.ShapeDtypeStruct((NC, n, 16), F32),
        ],
        mesh=mesh,
        compiler_params=pltpu.CompilerParams(
            use_tc_tiling_on_sc=False, needs_layout_passes=False),
        scratch_types=[
            pltpu.VMEM_SHARED((n, hc), F32),
            pltpu.VMEM_SHARED((n, 16), F32),
            pltpu.VMEM((2, CH), I32),
            pltpu.VMEM((2, CH), I32),
            pltpu.VMEM((2, CH), I32),
            pltpu.VMEM((2, CH, 16), F32),
            pltpu.VMEM((2, CH, 16), F32),
            pltpu.VMEM((2, CH, hc), F32),
            pltpu.VMEM((CH, hc), F32),
            pltpu.VMEM((2, CH, 16), F32),
        ] + [pltpu.SemaphoreType.DMA] * 10,
    )(h_arr, a_arr, src.reshape(NW, e_per_w), dst.reshape(NW, e_per_w))


def _run_combine(num_part, den_part, r16, bias):
    _, n, hc = num_part.shape
    bn = 1000
    return pl.pallas_call(
        _combine_body,
        grid=(n // bn,),
        in_specs=[
            pl.BlockSpec((NC, bn, hc), lambda i: (0, i, 0)),
            pl.BlockSpec((NC, bn, 16), lambda i: (0, i, 0)),
            pl.BlockSpec((16, hc), lambda i: (0, 0)),
            pl.BlockSpec((1, hc), lambda i: (0, 0)),
        ],
        out_specs=pl.BlockSpec((bn, hc), lambda i: (i, 0)),
        out_shape=jax.ShapeDtypeStruct((n, hc), F32),
    )(num_part, den_part, r16, bias.reshape(1, hc))


def kernel(x, edge_index, W, att_src, att_dst, bias):
    h_heads, c_dim = att_src.shape
    hc = h_heads * c_dim

    eye = jnp.eye(h_heads, dtype=F32)
    p_src = jnp.einsum("hc,hk->hck", att_src, eye).reshape(hc, h_heads)
    p_dst = jnp.einsum("hc,hk->hck", att_dst, eye).reshape(hc, h_heads)
    p = jnp.concatenate(
        [p_src, p_dst, jnp.zeros((hc, 16 - 2 * h_heads), F32)], axis=1)

    h_arr, a_arr = _run_proj(x, W, p)
    h_arr, a_arr, src, dst = lax.optimization_barrier(
        (h_arr, a_arr, edge_index[0], edge_index[1]))
    num_part, den_part = _run_edges(h_arr, a_arr, src, dst)
    num_part, den_part = lax.optimization_barrier((num_part, den_part))

    r16 = (jnp.arange(hc)[None, :] // c_dim
           == jnp.arange(16)[:, None]).astype(F32)
    return _run_combine(num_part, den_part, r16, bias)
